# baseline (speedup 1.0000x reference)
import jax
import jax.numpy as jnp
from jax.experimental import pallas as pl
from jax.experimental.pallas import tpu as pltpu

_B, _L, _C = 32, 2101, 64
_S = 64
_NB = 32
_W = _NB * _S
_PAD = _L - _W
_NBUF = 3
_BB = 4
_NS_STEPS = _B // _BB


def _compute(o, y):
    d = o - y
    sq = d * d
    main = sq[:_W].reshape(_NB, _S, _C)
    s = jnp.sum(main, axis=1)
    tail = jnp.sum(sq[_W:], axis=0, keepdims=True)
    mp = s[:-1] + s[1:]
    nz = (mp != 0).astype(jnp.float32)
    num = jnp.concatenate([mp[:1], mp[:-1] + mp[1:], mp[-1:]], axis=0)
    cnt = jnp.concatenate([nz[:1], nz[:-1] + nz[1:], nz[-1:]], axis=0)
    v = num / cnt
    body = jnp.broadcast_to(v[:, None, :], (_NB, _S, _C)).reshape(_W, _C)
    tail_b = jnp.broadcast_to(tail, (_PAD, _C))
    return jnp.concatenate([body, tail_b], axis=0)


def _man_kernel(o_hbm, y_hbm, out_hbm, obuf, ybuf, vbuf, isem, osem):
    def start_in(g, slot):
        sl = pl.ds(g * _BB, _BB)
        pltpu.make_async_copy(o_hbm.at[sl], obuf.at[slot], isem.at[slot, 0]).start()
        pltpu.make_async_copy(y_hbm.at[sl], ybuf.at[slot], isem.at[slot, 1]).start()

    for g in range(_NBUF):
        start_in(g, g)
    for g in range(_NS_STEPS):
        slot = g % _NBUF
        sl = pl.ds(g * _BB, _BB)
        pltpu.make_async_copy(o_hbm.at[sl], obuf.at[slot], isem.at[slot, 0]).wait()
        pltpu.make_async_copy(y_hbm.at[sl], ybuf.at[slot], isem.at[slot, 1]).wait()
        if g >= _NBUF:
            pltpu.make_async_copy(vbuf.at[slot], out_hbm.at[pl.ds((g - _NBUF) * _BB, _BB)],
                                  osem.at[slot]).wait()
        for bi in range(_BB):
            vbuf[slot, bi] = _compute(obuf[slot, bi], ybuf[slot, bi])
        pltpu.make_async_copy(vbuf.at[slot], out_hbm.at[sl], osem.at[slot]).start()
        if g + _NBUF < _NS_STEPS:
            start_in(g + _NBUF, slot)
    for g in range(_NS_STEPS - _NBUF, _NS_STEPS):
        slot = g % _NBUF
        pltpu.make_async_copy(vbuf.at[slot], out_hbm.at[pl.ds(g * _BB, _BB)], osem.at[slot]).wait()


def kernel(outputs, batch_y):
    return pl.pallas_call(
        _man_kernel,
        in_specs=[
            pl.BlockSpec(memory_space=pl.ANY),
            pl.BlockSpec(memory_space=pl.ANY),
        ],
        out_specs=pl.BlockSpec(memory_space=pl.ANY),
        out_shape=jax.ShapeDtypeStruct((_B, _L, _C), jnp.float32),
        scratch_shapes=[
            pltpu.VMEM((_NBUF, _BB, _L, _C), jnp.float32),
            pltpu.VMEM((_NBUF, _BB, _L, _C), jnp.float32),
            pltpu.VMEM((_NBUF, _BB, _L, _C), jnp.float32),
            pltpu.SemaphoreType.DMA((_NBUF, 2)),
            pltpu.SemaphoreType.DMA((_NBUF,)),
        ],
    )(outputs, batch_y)
